# baseline (speedup 1.0000x reference)
import functools

import jax
import jax.numpy as jnp
from jax import lax
from jax.experimental import pallas as pl
from jax.experimental.pallas import tpu as pltpu
from jax.experimental.pallas import tpu_sc as plsc

N_NODES = 10000
N_EDGES = 160000
DIM_IN = 256
DIM_HALF = 128
NUM_CORES = 2
NUM_SUBCORES = 16
CHUNK = 128
CHUNKS_PER_TILE = 80
HALF_CHUNKS = CHUNKS_PER_TILE // 2
N_CHUNKS = CHUNKS_PER_TILE * NUM_SUBCORES
E_PAD = N_CHUNKS * CHUNK
DUMMY_ROW = N_NODES
ACC_ROWS = N_NODES + 16
ROWS_PER_TILE = 624
ROWS_REM = N_NODES - ROWS_PER_TILE * NUM_SUBCORES
ZERO_REM = ACC_ROWS - ROWS_PER_TILE * NUM_SUBCORES


def _mm_body(x_ref, w_ref, o_ref):
    o_ref[...] = lax.dot_general(
        x_ref[...], w_ref[...], (((1,), (1,)), ((), ())),
        preferred_element_type=jnp.float32)


def _matmul_split(x, W):
    m_blk = 1000
    grid = (N_NODES // m_blk, NUM_CORES)
    return pl.pallas_call(
        _mm_body,
        grid=grid,
        in_specs=[
            pl.BlockSpec((m_blk, DIM_IN), lambda i, c: (i, 0)),
            pl.BlockSpec((DIM_HALF, DIM_IN), lambda i, c: (c, 0)),
        ],
        out_specs=pl.BlockSpec(
            (m_blk, DIM_HALF),
            lambda i, c: (c * (N_NODES // m_blk) + i, 0)),
        out_shape=jax.ShapeDtypeStruct((NUM_CORES * N_NODES, DIM_HALF),
                                       jnp.float32),
    )(x, W)


def _sc_aggregate(h2, src3, dst3, zeros):
    mesh = plsc.VectorSubcoreMesh(
        core_axis_name="c", subcore_axis_name="s",
        num_cores=NUM_CORES, num_subcores=NUM_SUBCORES)

    @functools.partial(
        pl.kernel,
        out_type=jax.ShapeDtypeStruct((NUM_CORES * N_NODES, DIM_HALF),
                                      jnp.float32),
        mesh=mesh,
        scratch_types=[
            pltpu.VMEM((HALF_CHUNKS, 1, CHUNK), jnp.int32),
            pltpu.VMEM((HALF_CHUNKS, 1, CHUNK), jnp.int32),
            pltpu.VMEM((2, CHUNK, DIM_HALF), jnp.float32),
            pltpu.VMEM_SHARED((ACC_ROWS, DIM_HALF), jnp.float32),
            pltpu.SemaphoreType.DMA,
            pltpu.SemaphoreType.DMA,
        ],
    )
    def agg(h_hbm, src_hbm, dst_hbm, z_hbm, out_hbm,
            sidx, didx, rows, acc, sem_g0, sem_g1):
        c = lax.axis_index("c")
        s = lax.axis_index("s")
        sems = (sem_g0, sem_g1)
        row0 = s * ROWS_PER_TILE
        pltpu.sync_copy(z_hbm.at[pl.ds(0, ROWS_PER_TILE)],
                        acc.at[pl.ds(row0, ROWS_PER_TILE)])

        @pl.when(s == NUM_SUBCORES - 1)
        def _():
            pltpu.sync_copy(
                z_hbm.at[pl.ds(0, ZERO_REM)],
                acc.at[pl.ds(ROWS_PER_TILE * NUM_SUBCORES, ZERO_REM)])

        plsc.subcore_barrier()

        h_view = h_hbm.at[pl.ds(c * N_NODES, N_NODES)]
        chunk0 = s * CHUNKS_PER_TILE

        def g_start(q, b):
            pltpu.async_copy(h_view.at[sidx.at[q, 0]], rows.at[b], sems[b])

        def g_drain(b):
            pltpu.make_async_copy(h_view.at[pl.ds(0, CHUNK)], rows.at[b],
                                  sems[b]).wait()

        def s_sync(q, b):
            pltpu.sync_copy(rows.at[b], acc.at[didx.at[q, 0]], add=True)

        def run_half(hf):
            base = chunk0 + hf * HALF_CHUNKS
            pltpu.sync_copy(src_hbm.at[pl.ds(base, HALF_CHUNKS)], sidx)
            pltpu.sync_copy(dst_hbm.at[pl.ds(base, HALF_CHUNKS)], didx)

            @pl.loop(0, HALF_CHUNKS)
            def _(q):
                pltpu.async_copy(h_view.at[sidx.at[q, 0]], rows.at[0],
                                 sem_g0).wait()
                s_sync(q, 0)

        run_half(0)
        run_half(1)

        plsc.subcore_barrier()
        pltpu.sync_copy(acc.at[pl.ds(row0, ROWS_PER_TILE)],
                        out_hbm.at[pl.ds(c * N_NODES + row0, ROWS_PER_TILE)])

        @pl.when(s == NUM_SUBCORES - 1)
        def _():
            tail0 = ROWS_PER_TILE * NUM_SUBCORES
            pltpu.sync_copy(acc.at[pl.ds(tail0, ROWS_REM)],
                            out_hbm.at[pl.ds(c * N_NODES + tail0, ROWS_REM)])

    return agg(h2, src3, dst3, zeros)


def kernel(x, edge_index, W):
    src = edge_index[0].astype(jnp.int32)
    dst = edge_index[1].astype(jnp.int32)
    pad = E_PAD - N_EDGES
    src3 = jnp.concatenate(
        [src, jnp.zeros((pad,), jnp.int32)]).reshape(N_CHUNKS, 1, CHUNK)
    dst3 = jnp.concatenate(
        [dst, jnp.full((pad,), DUMMY_ROW, jnp.int32)]).reshape(
            N_CHUNKS, 1, CHUNK)
    h2 = _matmul_split(x, W)
    zeros = jnp.zeros((ROWS_PER_TILE, DIM_HALF), jnp.float32)
    out2 = _sc_aggregate(h2, src3, dst3, zeros)
    return jnp.concatenate([out2[:N_NODES], out2[N_NODES:]], axis=1)


# speedup vs baseline: 1.2057x; 1.2057x over previous
import functools

import jax
import jax.numpy as jnp
from jax import lax
from jax.experimental import pallas as pl
from jax.experimental.pallas import tpu as pltpu
from jax.experimental.pallas import tpu_sc as plsc

N_NODES = 10000
N_EDGES = 160000
DIM_IN = 256
DIM_HALF = 128
NUM_CORES = 2
NUM_SUBCORES = 16
CHUNK = 128
CHUNKS_PER_TILE = 80
HALF_CHUNKS = CHUNKS_PER_TILE // 2
N_CHUNKS = CHUNKS_PER_TILE * NUM_SUBCORES
E_PAD = N_CHUNKS * CHUNK
DUMMY_ROW = N_NODES
ACC_ROWS = N_NODES + 16
ROWS_PER_TILE = 624
ROWS_REM = N_NODES - ROWS_PER_TILE * NUM_SUBCORES
ZERO_REM = ACC_ROWS - ROWS_PER_TILE * NUM_SUBCORES


def _mm_body(x_ref, w_ref, o_ref):
    o_ref[...] = lax.dot_general(
        x_ref[...], w_ref[...], (((1,), (1,)), ((), ())),
        preferred_element_type=jnp.float32)


def _matmul_split(x, W):
    m_blk = 1000
    grid = (N_NODES // m_blk, NUM_CORES)
    return pl.pallas_call(
        _mm_body,
        grid=grid,
        in_specs=[
            pl.BlockSpec((m_blk, DIM_IN), lambda i, c: (i, 0)),
            pl.BlockSpec((DIM_HALF, DIM_IN), lambda i, c: (c, 0)),
        ],
        out_specs=pl.BlockSpec(
            (m_blk, DIM_HALF),
            lambda i, c: (c * (N_NODES // m_blk) + i, 0)),
        out_shape=jax.ShapeDtypeStruct((NUM_CORES * N_NODES, DIM_HALF),
                                       jnp.float32),
    )(x, W)


def _sc_aggregate(h2, src3, zeros):
    mesh = plsc.VectorSubcoreMesh(
        core_axis_name="c", subcore_axis_name="s",
        num_cores=NUM_CORES, num_subcores=NUM_SUBCORES)

    @functools.partial(
        pl.kernel,
        out_type=jax.ShapeDtypeStruct((NUM_CORES * N_NODES, DIM_HALF),
                                      jnp.float32),
        mesh=mesh,
        scratch_types=[
            pltpu.VMEM((2, 1, 2, CHUNK), jnp.int32),
            pltpu.VMEM((2, CHUNK, DIM_HALF), jnp.float32),
            pltpu.VMEM_SHARED((ACC_ROWS, DIM_HALF), jnp.float32),
            pltpu.SemaphoreType.DMA,
            pltpu.SemaphoreType.DMA,
        ],
    )
    def agg(h_hbm, sd_hbm, z_hbm, out_hbm,
            sd, rows, acc, sem_g0, sem_g1):
        c = lax.axis_index("c")
        s = lax.axis_index("s")
        sems = (sem_g0, sem_g1)
        row0 = s * ROWS_PER_TILE
        pltpu.sync_copy(z_hbm.at[pl.ds(0, ROWS_PER_TILE)],
                        acc.at[pl.ds(row0, ROWS_PER_TILE)])

        @pl.when(s == NUM_SUBCORES - 1)
        def _():
            pltpu.sync_copy(
                z_hbm.at[pl.ds(0, ZERO_REM)],
                acc.at[pl.ds(ROWS_PER_TILE * NUM_SUBCORES, ZERO_REM)])

        plsc.subcore_barrier()

        off = c * N_NODES
        chunk0 = s * CHUNKS_PER_TILE

        def i_load(q, b):
            pltpu.sync_copy(sd_hbm.at[pl.ds(chunk0 + q, 1)], sd.at[b])

            @pl.loop(0, CHUNK, step=16)
            def _(k):
                sd[b, 0, 0, pl.ds(k, 16)] = sd[b, 0, 0, pl.ds(k, 16)] + off

        def g_start(b):
            pltpu.async_copy(h_hbm.at[sd.at[b, 0, 0]], rows.at[b], sems[b])

        def g_drain(b):
            pltpu.make_async_copy(h_hbm.at[pl.ds(0, CHUNK)], rows.at[b],
                                  sems[b]).wait()

        def s_sync(b):
            pltpu.sync_copy(rows.at[b], acc.at[sd.at[b, 0, 1]], add=True)

        def step(q, b, prefetch):
            if prefetch:
                i_load(q + 1, 1 - b)
                g_start(1 - b)
            g_drain(b)
            s_sync(b)

        def body(t, last):
            q0 = 2 * t
            step(q0, 0, True)
            step(q0 + 1, 1, not last)

        i_load(0, 0)
        g_start(0)
        body(0, False)

        @pl.loop(1, CHUNKS_PER_TILE // 2 - 1)
        def _(t):
            body(t, False)

        body(CHUNKS_PER_TILE // 2 - 1, True)

        plsc.subcore_barrier()
        pltpu.sync_copy(acc.at[pl.ds(row0, ROWS_PER_TILE)],
                        out_hbm.at[pl.ds(c * N_NODES + row0, ROWS_PER_TILE)])

        @pl.when(s == NUM_SUBCORES - 1)
        def _():
            tail0 = ROWS_PER_TILE * NUM_SUBCORES
            pltpu.sync_copy(acc.at[pl.ds(tail0, ROWS_REM)],
                            out_hbm.at[pl.ds(c * N_NODES + tail0, ROWS_REM)])

    return agg(h2, src3, zeros)


def kernel(x, edge_index, W):
    src = edge_index[0].astype(jnp.int32)
    dst = edge_index[1].astype(jnp.int32)
    pad = E_PAD - N_EDGES
    src2 = jnp.concatenate(
        [src, jnp.zeros((pad,), jnp.int32)]).reshape(N_CHUNKS, CHUNK)
    dst2 = jnp.concatenate(
        [dst, jnp.full((pad,), DUMMY_ROW, jnp.int32)]).reshape(N_CHUNKS, CHUNK)
    sd = jnp.stack([src2, dst2], axis=1)
    h2 = _matmul_split(x, W)
    zeros = jnp.zeros((ROWS_PER_TILE, DIM_HALF), jnp.float32)
    out2 = _sc_aggregate(h2, sd, zeros)
    return jnp.concatenate([out2[:N_NODES], out2[N_NODES:]], axis=1)
